# baseline (speedup 1.0000x reference)
import functools

import jax
import jax.numpy as jnp
from jax import lax
from jax.experimental import pallas as pl
from jax.experimental.pallas import tpu as pltpu
from jax.experimental.pallas import tpu_sc as plsc

_M, _D = 8192, 256
_NC, _NS, _L = 2, 16, 16
_NW = _NC * _NS
_RPW = _M // _NW
_NCHUNK = 4
_CH = _RPW // _NCHUNK


def _row_normalize_chunk(buf, slot):
    @plsc.parallel_loop(0, _CH, unroll=8)
    def _(r):
        vs = [buf[slot, r, pl.ds(j * _L, _L)] for j in range(_D // _L)]
        sq = [v * v for v in vs]
        while len(sq) > 1:
            sq = [sq[i] + sq[i + 1] for i in range(0, len(sq), 2)]
        ss = jnp.maximum(jnp.sum(sq[0]), 1e-24)
        x = jnp.full((_L,), ss, dtype=jnp.float32)
        i = plsc.bitcast(x, jnp.int32)
        y = plsc.bitcast(jnp.int32(0x5F3759DF) - (i >> 1), jnp.float32)
        for _ in range(3):
            y = y * (1.5 - 0.5 * x * y * y)
        for j in range(_D // _L):
            buf[slot, r, pl.ds(j * _L, _L)] = vs[j] * y


def _sc_norm_body(x_hbm, o_hbm, buf, in_sems, out_sems):
    wid = lax.axis_index("s") * _NC + lax.axis_index("c")
    base = wid * _RPW
    for k in range(_NCHUNK):
        pltpu.make_async_copy(
            x_hbm.at[pl.ds(base + k * _CH, _CH), :], buf.at[k], in_sems.at[k]
        ).start()
    for k in range(_NCHUNK):
        pltpu.make_async_copy(
            x_hbm.at[pl.ds(base + k * _CH, _CH), :], buf.at[k], in_sems.at[k]
        ).wait()
        _row_normalize_chunk(buf, k)
        pltpu.make_async_copy(
            buf.at[k], o_hbm.at[pl.ds(base + k * _CH, _CH), :], out_sems.at[k]
        ).start()
    for k in range(_NCHUNK):
        pltpu.make_async_copy(
            buf.at[k], o_hbm.at[pl.ds(base + k * _CH, _CH), :], out_sems.at[k]
        ).wait()


def kernel(prototypes):
    sc_norm = functools.partial(
        pl.kernel,
        out_type=jax.ShapeDtypeStruct((_M, _D), jnp.float32),
        mesh=plsc.VectorSubcoreMesh(
            core_axis_name="c", subcore_axis_name="s",
            num_cores=_NC, num_subcores=_NS,
        ),
        scratch_types=[
            pltpu.VMEM((_NCHUNK, _CH, _D), jnp.float32),
            pltpu.SemaphoreType.DMA((_NCHUNK,)),
            pltpu.SemaphoreType.DMA((_NCHUNK,)),
        ],
        compiler_params=pltpu.CompilerParams(needs_layout_passes=False),
    )(_sc_norm_body)
    return sc_norm(prototypes)


# speedup vs baseline: 5.1239x; 5.1239x over previous
import jax
import jax.numpy as jnp
from jax.experimental import pallas as pl
from jax.experimental.pallas import tpu as pltpu

_M, _D = 8192, 256
_NCH = 8
_CH = _M // _NCH


def _norm_pipeline(x_hbm, x_hbm2, o_hbm, vin, vout, in_sems, out_sems):
    srcs = [x_hbm, x_hbm2]
    for i in range(_NCH):
        pltpu.make_async_copy(
            srcs[i % 2].at[pl.ds(i * _CH, _CH), :], vin.at[i], in_sems.at[i]
        ).start()
    for i in range(_NCH):
        pltpu.make_async_copy(
            srcs[i % 2].at[pl.ds(i * _CH, _CH), :], vin.at[i], in_sems.at[i]
        ).wait()
        x = vin[i]
        ss = jnp.sum(x * x, axis=-1, keepdims=True)
        vout[i] = x * jax.lax.rsqrt(jnp.maximum(ss, 1e-24))
        pltpu.make_async_copy(
            vout.at[i], o_hbm.at[pl.ds(i * _CH, _CH), :], out_sems.at[i]
        ).start()
    for i in range(_NCH):
        pltpu.make_async_copy(
            vout.at[i], o_hbm.at[pl.ds(i * _CH, _CH), :], out_sems.at[i]
        ).wait()


def kernel(prototypes):
    return pl.pallas_call(
        _norm_pipeline,
        in_specs=[pl.BlockSpec(memory_space=pl.ANY),
                  pl.BlockSpec(memory_space=pl.ANY)],
        out_specs=pl.BlockSpec(memory_space=pl.ANY),
        out_shape=jax.ShapeDtypeStruct((_M, _D), prototypes.dtype),
        scratch_shapes=[
            pltpu.VMEM((_NCH, _CH, _D), jnp.float32),
            pltpu.VMEM((_NCH, _CH, _D), jnp.float32),
            pltpu.SemaphoreType.DMA((_NCH,)),
            pltpu.SemaphoreType.DMA((_NCH,)),
        ],
    )(prototypes, prototypes)


# speedup vs baseline: 5.1921x; 1.0133x over previous
import jax
import jax.numpy as jnp
from jax.experimental import pallas as pl
from jax.experimental.pallas import tpu as pltpu

_M, _D = 8192, 256
_NCH = 8
_CH = _M // _NCH


def _norm_pipeline(x_hbm, o_hbm, vin, vout, in_sems, out_sems):
    for i in range(_NCH):
        pltpu.make_async_copy(
            x_hbm.at[pl.ds(i * _CH, _CH), :], vin.at[i], in_sems.at[i]
        ).start()
    for i in range(_NCH):
        pltpu.make_async_copy(
            x_hbm.at[pl.ds(i * _CH, _CH), :], vin.at[i], in_sems.at[i]
        ).wait()
        x = vin[i]
        ss = jnp.sum(x * x, axis=-1, keepdims=True)
        vout[i] = x * jax.lax.rsqrt(jnp.maximum(ss, 1e-24))
        pltpu.make_async_copy(
            vout.at[i], o_hbm.at[pl.ds(i * _CH, _CH), :], out_sems.at[i]
        ).start()
    for i in range(_NCH):
        pltpu.make_async_copy(
            vout.at[i], o_hbm.at[pl.ds(i * _CH, _CH), :], out_sems.at[i]
        ).wait()


def kernel(prototypes):
    return pl.pallas_call(
        _norm_pipeline,
        in_specs=[pl.BlockSpec(memory_space=pl.ANY)],
        out_specs=pl.BlockSpec(memory_space=pl.ANY),
        out_shape=jax.ShapeDtypeStruct((_M, _D), prototypes.dtype),
        scratch_shapes=[
            pltpu.VMEM((_NCH, _CH, _D), jnp.float32),
            pltpu.VMEM((_NCH, _CH, _D), jnp.float32),
            pltpu.SemaphoreType.DMA((_NCH,)),
            pltpu.SemaphoreType.DMA((_NCH,)),
        ],
    )(prototypes)
